# baseline (speedup 1.0000x reference)
import functools

import jax
import jax.numpy as jnp
from jax import lax
from jax.experimental import pallas as pl
from jax.experimental.pallas import tpu as pltpu
from jax.experimental.pallas import tpu_sc as plsc

B, N, D = 1024, 77, 768
NUM_NAMES = 1000
PLACEHOLDER_TOKEN = 265

_NC, _NS = 2, 16
_NW = _NC * _NS
_RPW = B // _NW
_L = 16
_OFFS = (0, 16, 32, 48, 61)

_GDN = lax.GatherDimensionNumbers(
    offset_dims=(), collapsed_slice_dims=(0,), start_index_map=(0,)
)


def _rot(v, s):
    perm = ((lax.iota(jnp.int32, _L) + s) % _L).reshape(_L, 1)
    return lax.gather(
        v, perm, _GDN, slice_sizes=(1,),
        mode=lax.GatherScatterMode.PROMISE_IN_BOUNDS,
    )


def _sc_body(tok_ref, name_ref, params_ref, g_out, c_out, idx_v, col_v,
             rows_v, tokv, sem):
    wid = lax.axis_index("s") * _NC + lax.axis_index("c")
    base = wid * _RPW

    pltpu.sync_copy(name_ref.at[pl.ds(base, _RPW)], idx_v)
    pltpu.async_copy(params_ref.at[idx_v], rows_v, sem).wait()
    pltpu.sync_copy(rows_v, g_out.at[pl.ds(base, _RPW)])

    pltpu.sync_copy(tok_ref.at[pl.ds(base, _RPW)], tokv)

    lanes = lax.iota(jnp.int32, _L)
    for g in range(_RPW // _L):
        merged = jnp.zeros((_L,), jnp.int32)
        for q in range(_L):
            r = g * _L + q
            col = jnp.zeros((_L,), jnp.int32)
            for off in _OFFS:
                m = tokv[r, pl.ds(off, _L)] == PLACEHOLDER_TOKEN
                if off == 61:
                    m = jnp.logical_and(m, lanes >= 3)
                col = col + jnp.where(m, off + lanes, 0)
            for s in (8, 4, 2, 1):
                col = col + _rot(col, s)
            merged = jnp.where(lanes == q, col, merged)
        col_v[pl.ds(g * _L, _L)] = merged
    pltpu.sync_copy(col_v, c_out.at[pl.ds(base, _RPW)])


@functools.cache
def _sc_gather_cols():
    return pl.kernel(
        _sc_body,
        out_type=(
            jax.ShapeDtypeStruct((B, D), jnp.float32),
            jax.ShapeDtypeStruct((B,), jnp.int32),
        ),
        mesh=plsc.VectorSubcoreMesh(core_axis_name="c", subcore_axis_name="s"),
        scratch_types=[
            pltpu.VMEM((_RPW,), jnp.int32),
            pltpu.VMEM((_RPW,), jnp.int32),
            pltpu.VMEM((_RPW, D), jnp.float32),
            pltpu.VMEM((_RPW, N), jnp.int32),
            pltpu.SemaphoreType.DMA,
        ],
    )


_CHUNK = 256


def _tc_scatter_body(c_ref, g_ref, g_hbm, emb_ref, out_ref, sem):
    def chunk(ci, _):
        def row(i, _):
            col = c_ref[i]
            pltpu.make_async_copy(
                g_ref.at[pl.ds(i, 1)],
                out_ref.at[i, pl.ds(col, 1)],
                sem,
            ).start()
            return 0

        lax.fori_loop(ci * _CHUNK, (ci + 1) * _CHUNK, row, 0)
        pltpu.make_async_copy(
            g_hbm.at[pl.ds(0, _CHUNK)], g_ref.at[pl.ds(0, _CHUNK)], sem
        ).wait()
        return 0

    lax.fori_loop(0, B // _CHUNK, chunk, 0)


def _tc_scatter(cols, gathered, embedded_text):
    return pl.pallas_call(
        _tc_scatter_body,
        in_specs=[
            pl.BlockSpec(memory_space=pltpu.SMEM),
            pl.BlockSpec(memory_space=pltpu.VMEM),
            pl.BlockSpec(memory_space=pl.ANY),
            pl.BlockSpec(memory_space=pl.ANY),
        ],
        out_specs=pl.BlockSpec(memory_space=pl.ANY),
        out_shape=jax.ShapeDtypeStruct((B, N, D), jnp.float32),
        scratch_shapes=[pltpu.SemaphoreType.DMA],
        input_output_aliases={3: 0},
    )(cols, gathered, gathered, embedded_text)


def kernel(tokenized_text, embedded_text, name, params):
    params2d = params.reshape(NUM_NAMES, D)
    gathered, cols = _sc_gather_cols()(tokenized_text, name, params2d)
    zero = jnp.minimum(name[0], 0).astype(jnp.float32)
    return _tc_scatter(cols, gathered, embedded_text + zero)


# speedup vs baseline: 2.1105x; 2.1105x over previous
import functools

import jax
import jax.numpy as jnp
from jax import lax
from jax.experimental import pallas as pl
from jax.experimental.pallas import tpu as pltpu
from jax.experimental.pallas import tpu_sc as plsc

B, N, D = 1024, 77, 768
NUM_NAMES = 1000
PLACEHOLDER_TOKEN = 265

_NC, _NS = 2, 16
_NW = _NC * _NS
_RPW = B // _NW
_L = 16
_OFFS = (0, 16, 32, 48, 61)

_GDN = lax.GatherDimensionNumbers(
    offset_dims=(), collapsed_slice_dims=(0,), start_index_map=(0,)
)


def _rot(v, s):
    perm = ((lax.iota(jnp.int32, _L) + s) % _L).reshape(_L, 1)
    return lax.gather(
        v, perm, _GDN, slice_sizes=(1,),
        mode=lax.GatherScatterMode.PROMISE_IN_BOUNDS,
    )


def _sc_body(tok_ref, name_ref, params_ref, g_out, c_out, idx_v, col_v,
             rows_v, tokv, sem):
    wid = lax.axis_index("s") * _NC + lax.axis_index("c")
    base = wid * _RPW

    pltpu.sync_copy(name_ref.at[pl.ds(base, _RPW)], idx_v)
    pltpu.async_copy(params_ref.at[idx_v], rows_v, sem).wait()
    pltpu.sync_copy(rows_v, g_out.at[pl.ds(base, _RPW)])

    pltpu.sync_copy(tok_ref.at[pl.ds(base, _RPW)], tokv)

    lanes = lax.iota(jnp.int32, _L)
    for g in range(_RPW // _L):
        merged = jnp.zeros((_L,), jnp.int32)
        for q in range(_L):
            r = g * _L + q
            col = jnp.zeros((_L,), jnp.int32)
            for off in _OFFS:
                m = tokv[r, pl.ds(off, _L)] == PLACEHOLDER_TOKEN
                if off == 61:
                    m = jnp.logical_and(m, lanes >= 3)
                col = col + jnp.where(m, off + lanes, 0)
            for s in (8, 4, 2, 1):
                col = col + _rot(col, s)
            merged = jnp.where(lanes == q, col, merged)
        col_v[pl.ds(g * _L, _L)] = merged
    pltpu.sync_copy(col_v, c_out.at[pl.ds(base, _RPW)])


@functools.cache
def _sc_gather_cols():
    return pl.kernel(
        _sc_body,
        out_type=(
            jax.ShapeDtypeStruct((B, D), jnp.float32),
            jax.ShapeDtypeStruct((B,), jnp.int32),
        ),
        mesh=plsc.VectorSubcoreMesh(core_axis_name="c", subcore_axis_name="s"),
        scratch_types=[
            pltpu.VMEM((_RPW,), jnp.int32),
            pltpu.VMEM((_RPW,), jnp.int32),
            pltpu.VMEM((_RPW, D), jnp.float32),
            pltpu.VMEM((_RPW, N), jnp.int32),
            pltpu.SemaphoreType.DMA,
        ],
    )


_CHUNK = 256


def _tc_scatter_body(c_ref, g_ref, g_hbm, emb_ref, out_ref, sem):
    def chunk(ci, _):
        def row(i, _):
            col = c_ref[i]
            pltpu.make_async_copy(
                g_ref.at[pl.ds(i, 1)],
                out_ref.at[col, pl.ds(i, 1)],
                sem,
            ).start()
            return 0

        lax.fori_loop(ci * _CHUNK, (ci + 1) * _CHUNK, row, 0)
        pltpu.make_async_copy(
            g_hbm.at[pl.ds(0, _CHUNK)], g_ref.at[pl.ds(0, _CHUNK)], sem
        ).wait()
        return 0

    lax.fori_loop(0, B // _CHUNK, chunk, 0)


def _tc_scatter(cols, gathered, embedded_text):
    return pl.pallas_call(
        _tc_scatter_body,
        in_specs=[
            pl.BlockSpec(memory_space=pltpu.SMEM),
            pl.BlockSpec(memory_space=pltpu.VMEM),
            pl.BlockSpec(memory_space=pl.ANY),
            pl.BlockSpec(memory_space=pl.ANY),
        ],
        out_specs=pl.BlockSpec(memory_space=pl.ANY),
        out_shape=jax.ShapeDtypeStruct((N, B, D), jnp.float32),
        scratch_shapes=[pltpu.SemaphoreType.DMA],
        input_output_aliases={3: 0},
    )(cols, gathered, gathered, embedded_text)


def kernel(tokenized_text, embedded_text, name, params):
    params2d = params.reshape(NUM_NAMES, D)
    gathered, cols = _sc_gather_cols()(tokenized_text, name, params2d)
    zero = jnp.minimum(name[0], 0).astype(jnp.float32)
    emb_t = jnp.transpose(embedded_text, (1, 0, 2)) + zero
    out_t = _tc_scatter(cols, gathered, emb_t)
    return jnp.transpose(out_t, (1, 0, 2))
